# baseline (speedup 1.0000x reference)
import functools

import jax
import jax.numpy as jnp
import numpy as np
from jax import lax
from jax.experimental import pallas as pl
from jax.experimental.pallas import tpu as pltpu
from jax.experimental.pallas import tpu_sc as plsc

_CARDS = [100000] * 26
_N_CAT = 26
_N_NUM = 13
_D = 64
_BATCH = 16384
_TOTAL_ROWS = sum(_CARDS)

_NC, _NS = 2, 16
_NW = _NC * _NS
_RPT = _BATCH // _NW
_CB = 16
_NCHUNK = _RPT // _CB
_IDX_PER_CHUNK = _CB * _N_CAT
_GATHER_IDX = 104
_NGATHER = _IDX_PER_CHUNK // _GATHER_IDX
_ROW_OUT = (_N_NUM + _N_CAT) * _D
_CHUNK_OUT = _CB * _ROW_OUT
_L = 16


def _body(x_num_h, x_cat_h, w_h, b_h, bias_h, off_h, table_h, out_h,
          xnum_v, idx_v, w_v, b_v, bias_v, off_v, rows_v, out_v, gsem):
    wid = lax.axis_index("s") * _NC + lax.axis_index("c")

    pltpu.sync_copy(x_num_h.at[pl.ds(wid * (_RPT * _N_NUM), _RPT * _N_NUM)],
                    xnum_v.at[pl.ds(0, _RPT * _N_NUM)])
    pltpu.sync_copy(x_cat_h.at[pl.ds(wid * (_RPT * _N_CAT), _RPT * _N_CAT)],
                    idx_v)
    pltpu.sync_copy(w_h, w_v)
    pltpu.sync_copy(b_h, b_v)
    pltpu.sync_copy(bias_h, bias_v)
    pltpu.sync_copy(off_h, off_v)

    def add_off(j, carry):
        for k in range(_N_NUM):
            m = (j * 13 + k) * _L
            idx_v[pl.ds(m, _L)] = idx_v[pl.ds(m, _L)] + off_v[pl.ds(k * _L, _L)]
        return carry
    lax.fori_loop(0, _RPT * _N_CAT // (13 * _L), add_off, 0)

    def chunk(c, carry):
        ib = c * _IDX_PER_CHUNK
        copies = []
        for g in range(_NGATHER):
            copies.append(pltpu.async_copy(
                table_h.at[idx_v.at[pl.ds(ib + g * _GATHER_IDX, _GATHER_IDX)]],
                rows_v.at[pl.ds(g * _GATHER_IDX, _GATHER_IDX), :],
                gsem))

        def num_row(r, inner):
            rowbase = r * _ROW_OUT
            xv = xnum_v[pl.ds((c * _CB + r) * _N_NUM, _L)]
            for f in range(_N_NUM):
                s = xv[f]
                for d in range(_D // _L):
                    o = f * _D + d * _L
                    out_v[pl.ds(rowbase + o, _L)] = (
                        s * w_v[pl.ds(o, _L)] + b_v[pl.ds(o, _L)])
            return inner
        lax.fori_loop(0, _CB, num_row, 0)

        for cp in copies:
            cp.wait()

        def cat_row(r, inner):
            rowbase = r * _ROW_OUT + _N_NUM * _D
            for j in range(_N_CAT):
                rr = r * _N_CAT + j
                for d in range(_D // _L):
                    out_v[pl.ds(rowbase + j * _D + d * _L, _L)] = (
                        rows_v[rr, pl.ds(d * _L, _L)]
                        + bias_v[j, pl.ds(d * _L, _L)])
            return inner
        lax.fori_loop(0, _CB, cat_row, 0)

        pltpu.sync_copy(out_v,
                        out_h.at[pl.ds((wid * _RPT + c * _CB) * _ROW_OUT,
                                       _CHUNK_OUT)])
        return carry
    lax.fori_loop(0, _NCHUNK, chunk, 0)


def kernel(x_num, x_cat, W_num, b_num, table, cat_bias):
    offsets = np.concatenate([[0], np.cumsum(_CARDS[:-1])]).astype(np.int32)
    off_tiled = jnp.asarray(np.tile(offsets, 8))

    mesh = plsc.VectorSubcoreMesh(core_axis_name="c", subcore_axis_name="s")
    run = pl.kernel(
        _body,
        out_type=jax.ShapeDtypeStruct((_BATCH * _ROW_OUT,), jnp.float32),
        mesh=mesh,
        compiler_params=pltpu.CompilerParams(use_tc_tiling_on_sc=False),
        scratch_types=[
            pltpu.VMEM((_RPT * _N_NUM + _L,), jnp.float32),
            pltpu.VMEM((_RPT * _N_CAT,), jnp.int32),
            pltpu.VMEM((_N_NUM * _D,), jnp.float32),
            pltpu.VMEM((_N_NUM * _D,), jnp.float32),
            pltpu.VMEM((_N_CAT, _D), jnp.float32),
            pltpu.VMEM((13 * _L,), jnp.int32),
            pltpu.VMEM((_IDX_PER_CHUNK, _D), jnp.float32),
            pltpu.VMEM((_CHUNK_OUT,), jnp.float32),
            pltpu.SemaphoreType.DMA,
        ],
    )
    out = run(x_num.reshape(-1), x_cat.reshape(-1),
              W_num.reshape(-1), b_num.reshape(-1),
              cat_bias, off_tiled, table)
    return out.reshape(_BATCH, _N_NUM + _N_CAT, _D)
